# baseline (speedup 1.0000x reference)
import functools

import jax
import jax.numpy as jnp
from jax import lax
from jax.experimental import pallas as pl
from jax.experimental.pallas import tpu as pltpu
from jax.experimental.pallas import tpu_sc as plsc

N_USERS = 100000
D_BIAS = 64
BATCH = 4096

_INFO = plsc.get_sparse_core_info()
_NC = _INFO.num_cores
_NS = _INFO.num_subcores
_NL = _INFO.num_lanes
_NW = _NC * _NS
_ROWS_PER_W = D_BIAS // _NW


@functools.partial(
    pl.kernel,
    mesh=plsc.VectorSubcoreMesh(core_axis_name="c", subcore_axis_name="s"),
    out_type=jax.ShapeDtypeStruct((D_BIAS, BATCH), jnp.float32),
    scratch_types=[
        pltpu.VMEM((BATCH,), jnp.int32),
        pltpu.VMEM((N_USERS,), jnp.float32),
        pltpu.VMEM((_ROWS_PER_W, BATCH), jnp.float32),
        pltpu.SemaphoreType.DMA,
        pltpu.SemaphoreType.DMA,
        pltpu.SemaphoreType.DMA,
    ],
    compiler_params=pltpu.CompilerParams(needs_layout_passes=False),
)
def _sc_gather_t(
    wt_hbm, idx_hbm, out_hbm, idx_v, row_v, out_v, sem_i, sem_r, sem_o
):
    wid = lax.axis_index("s") * _NC + lax.axis_index("c")
    cp_idx = pltpu.make_async_copy(idx_hbm, idx_v, sem_i)
    cp_idx.start()
    d0 = wid * _ROWS_PER_W
    cp_row0 = pltpu.make_async_copy(wt_hbm.at[d0], row_v, sem_r)
    cp_row0.start()
    cp_idx.wait()

    out_cps = []
    for r in range(_ROWS_PER_W):
        d = wid * _ROWS_PER_W + r
        cp_row = pltpu.make_async_copy(wt_hbm.at[d], row_v, sem_r)
        if r > 0:
            cp_row.start()
        cp_row.wait()

        def body(g, carry):
            idx16 = idx_v[pl.ds(g * _NL, _NL)]
            out_v[r, pl.ds(g * _NL, _NL)] = plsc.load_gather(row_v, [idx16])
            return carry

        lax.fori_loop(0, BATCH // _NL, body, 0, unroll=8)
        cp_out = pltpu.make_async_copy(out_v.at[r], out_hbm.at[d], sem_o)
        cp_out.start()
        out_cps.append(cp_out)
    for cp_out in out_cps:
        cp_out.wait()


def kernel(user_id, weight):
    out_t = _sc_gather_t(weight.T, user_id.astype(jnp.int32))
    return out_t.T
